# baseline (speedup 1.0000x reference)
import functools

import jax
import jax.numpy as jnp
from jax import lax
from jax.experimental import pallas as pl
from jax.experimental.pallas import tpu as pltpu
from jax.experimental.pallas import tpu_sc as plsc

N_NODES = 50000
T = 3
D = 128
M = 128
L11 = 64
B = 1024
S1 = 10
S2 = 25
SLOTS = 1 + S1 + S1 * S2
SLOTS_PAD = 264
EROW = T * L11
FROW = T * D

NW = 32
TGT_PER_W = B // NW
PER_W = TGT_PER_W * SLOTS_PAD
CHUNK = 128
NCHUNK = PER_W // CHUNK

PROJ_BN = 3000
BB = 16


def _proj_body(f_ref, v_ref, o_ref):
    o_ref[...] = jnp.dot(f_ref[...], v_ref[...],
                         preferred_element_type=jnp.float32)


def _softmax(x):
    m = jnp.max(x, axis=-1, keepdims=True)
    e = jnp.exp(x - m)
    return e / jnp.sum(e, axis=-1, keepdims=True)


def _leaky(x):
    return jnp.where(x >= 0, x, 0.01 * x)


def _attn_gru_body(g_ref, xg_ref, wab_ref, w0a_ref, w0b_ref,
                   wha_ref, whb_ref, v0t_ref, wz1_ref, wz2_ref, wz3_ref,
                   wr11_ref, wr12_ref, wr13_ref, wr21t_ref,
                   wr22at_ref, wr22bt_ref, wh1t_ref, wh2at_ref, wh2bt_ref,
                   bz_ref, br1_ref, br2_ref, bh_ref, o_ref, h_ref):
    t = pl.program_id(1)
    et = g_ref[...]
    xt = xg_ref[...]
    w0a = w0a_ref[...]
    w0b = w0b_ref[...]
    wha = wha_ref[...]
    whb = whb_ref[...]
    v0t = v0t_ref[...]

    @pl.when(t == 0)
    def _():
        h_ref[...] = jnp.zeros((BB, M), jnp.float32)

    h = h_ref[...]

    ab3 = jax.lax.dot_general(
        et, wab_ref[...], (((2,), (0,)), ((), ())),
        preferred_element_type=jnp.float32)
    a_s = ab3[:, :, 0]
    b_s = ab3[:, :, 1]

    e_tgt = et[:, 0, :]
    e_h1 = et[:, 1:1 + S1, :]
    e_h2r = et[:, 1 + S1:1 + S1 + S1 * S2, :].reshape(BB, S1, S2, L11)

    a0 = a_s[:, 0]
    a1 = a_s[:, 1:1 + S1]
    b1 = b_s[:, 1:1 + S1]
    s2d = b_s[:, 1 + S1:1 + S1 + S1 * S2].reshape(BB, S1, S2)

    beta1 = _softmax(_leaky(a1[:, :, None] + s2d))
    agg1 = jnp.sum(beta1[..., None] * e_h2r, axis=2)
    e_h1f = e_h1.reshape(BB * S1, L11)
    agg1f = agg1.reshape(BB * S1, L11)
    hop1f = jax.nn.sigmoid(
        jnp.dot(e_h1f, wha, preferred_element_type=jnp.float32)
        + jnp.dot(agg1f, whb, preferred_element_type=jnp.float32))

    beta0 = _softmax(_leaky(a0[:, None] + b1))
    agg0 = jnp.sum(beta0[..., None] * e_h1, axis=1)
    hop0 = jax.nn.sigmoid(
        jnp.dot(e_tgt, wha, preferred_element_type=jnp.float32)
        + jnp.dot(agg0, whb, preferred_element_type=jnp.float32))

    p0 = jnp.dot(hop0, v0t, preferred_element_type=jnp.float32)
    p1f = jnp.dot(hop1f, v0t, preferred_element_type=jnp.float32)
    p1 = p1f.reshape(BB, S1, M)
    c0 = jnp.sum(p0 * w0a, axis=-1)
    c1 = jnp.sum(p1 * w0b[:, None, :], axis=-1)
    betac = _softmax(_leaky(c0[:, None] + c1))
    xnt = jnp.sum(betac[..., None] * p1, axis=1)

    z = jax.nn.sigmoid(
        jnp.dot(h, wz1_ref[...], preferred_element_type=jnp.float32)
        + jnp.dot(xt, wz2_ref[...], preferred_element_type=jnp.float32)
        + jnp.dot(xnt, wz3_ref[...], preferred_element_type=jnp.float32)
        + bz_ref[...])
    r1 = jax.nn.sigmoid(
        jnp.dot(h, wr11_ref[...], preferred_element_type=jnp.float32)
        + jnp.dot(xt, wr12_ref[...], preferred_element_type=jnp.float32)
        + jnp.dot(xnt, wr13_ref[...], preferred_element_type=jnp.float32)
        + br1_ref[...])
    r2 = jax.nn.sigmoid(
        jnp.dot(h, wr21t_ref[...], preferred_element_type=jnp.float32)
        + jnp.dot(xt, wr22at_ref[...], preferred_element_type=jnp.float32)
        + jnp.dot(xnt, wr22bt_ref[...], preferred_element_type=jnp.float32)
        + br2_ref[...])
    h_til = jnp.tanh(
        jnp.dot(r1 * h, wh1t_ref[...], preferred_element_type=jnp.float32)
        + jnp.dot(xt, wh2at_ref[...], preferred_element_type=jnp.float32)
        + jnp.dot(r2 * xnt, wh2bt_ref[...],
                  preferred_element_type=jnp.float32)
        + bh_ref[...])
    h = (1.0 - z) * h + z * h_til
    h_ref[...] = h
    o_ref[...] = h


def _full(shape):
    return pl.BlockSpec(shape, lambda i, t: (0,) * len(shape))


def _attn_specs():
    in_specs = [
        pl.BlockSpec((BB, SLOTS_PAD, L11),
                     lambda i, t: (t * (B // BB) + i, 0, 0)),
        pl.BlockSpec((BB, D), lambda i, t: (t * (B // BB) + i, 0)),
        _full((L11, 2)),
        _full((1, M)), _full((1, M)),
        _full((L11, L11)), _full((L11, L11)),
        _full((L11, M)),
        _full((M, M)), _full((D, M)), _full((D, M)),
        _full((M, M)), _full((D, M)), _full((D, M)),
        _full((M, M)), _full((D, M)), _full((D, M)),
        _full((M, M)), _full((D, M)), _full((D, M)),
        _full((1, M)), _full((1, M)), _full((1, D)), _full((1, M)),
    ]
    out_specs = pl.BlockSpec((BB, M), lambda i, t: (i, 0))
    return in_specs, out_specs


def _sc_gather(e50, fea50, idx2, xidx2):
    try:
        info = plsc.get_sparse_core_info()
        nc, ns = info.num_cores, info.num_subcores
    except Exception:
        nc, ns = 2, 16
    mesh = plsc.VectorSubcoreMesh(core_axis_name="c", subcore_axis_name="s")

    @functools.partial(
        pl.kernel,
        out_type=(jax.ShapeDtypeStruct((T, B * SLOTS_PAD, L11), jnp.float32),
                  jax.ShapeDtypeStruct((T, B, D), jnp.float32)),
        mesh=mesh,
        scratch_types=[
            pltpu.VMEM((PER_W,), jnp.int32),
            pltpu.VMEM((CHUNK, EROW), jnp.float32),
            pltpu.VMEM((CHUNK, EROW), jnp.float32),
            pltpu.VMEM((TGT_PER_W,), jnp.int32),
            pltpu.VMEM((TGT_PER_W, FROW), jnp.float32),
            pltpu.SemaphoreType.DMA,
        ],
        compiler_params=pltpu.CompilerParams(use_tc_tiling_on_sc=False),
    )
    def k(e_hbm, fea_hbm, idx_hbm, xidx_hbm, g_hbm, xg_hbm,
          idx_v, buf0, buf1, xidx_v, xrows_v, sem):
        wid = lax.axis_index("s") * nc + lax.axis_index("c")
        pltpu.sync_copy(idx_hbm.at[wid], idx_v)
        pltpu.sync_copy(xidx_hbm.at[wid], xidx_v)
        pltpu.async_copy(fea_hbm.at[xidx_v], xrows_v, sem).wait()
        xoff = pl.multiple_of(wid * TGT_PER_W, 8)
        for tt in range(T):
            pltpu.sync_copy(xrows_v.at[:, pl.ds(tt * D, D)],
                            xg_hbm.at[tt, pl.ds(xoff, TGT_PER_W)])
        base = wid * PER_W
        bufs = (buf0, buf1)

        def fire(c, buf):
            off = pl.multiple_of(c * CHUNK, 8)
            pltpu.async_copy(e_hbm.at[idx_v.at[pl.ds(off, CHUNK)]], buf, sem)

        def drain(buf):
            pltpu.make_async_copy(e_hbm.at[pl.ds(0, CHUNK)], buf, sem).wait()

        fire(0, buf0)

        @pl.loop(0, NCHUNK, step=2)
        def _(j):
            for b in range(2):
                cur = j + b

                @pl.when(cur + 1 < NCHUNK)
                def _():
                    fire(cur + 1, bufs[1 - b])

                drain(bufs[b])
                off = pl.multiple_of(base + cur * CHUNK, 8)
                for tt in range(T):
                    pltpu.sync_copy(bufs[b].at[:, pl.ds(tt * L11, L11)],
                                    g_hbm.at[tt, pl.ds(off, CHUNK)])

    return k(e50, fea50, idx2, xidx2)


def kernel(x, saps_idx, Fea, V1_h1, w1_h1, V1_h0, w1_h0, weights_hops_1,
           Wz, Wr1, Wr2_1, Wr2_2, Wh_1, Wh_2, bz, br1, br2, bh):
    f32 = jnp.float32
    fea2 = Fea.reshape(N_NODES * T, D)

    e_all = pl.pallas_call(
        _proj_body,
        grid=(N_NODES * T // PROJ_BN,),
        in_specs=[pl.BlockSpec((PROJ_BN, D), lambda i: (i, 0)),
                  pl.BlockSpec((D, L11), lambda i: (0, 0))],
        out_specs=pl.BlockSpec((PROJ_BN, L11), lambda i: (i, 0)),
        out_shape=jax.ShapeDtypeStruct((N_NODES * T, L11), f32),
    )(fea2, V1_h1.T.astype(f32))
    e50 = e_all.reshape(N_NODES, EROW)
    fea50 = Fea.reshape(N_NODES, FROW)

    idx_pad = jnp.concatenate(
        [saps_idx.astype(jnp.int32),
         jnp.zeros((B, SLOTS_PAD - SLOTS), jnp.int32)], axis=1)
    idx2 = idx_pad.reshape(NW, PER_W)
    xidx2 = x.astype(jnp.int32).reshape(NW, TGT_PER_W)
    g, xg = _sc_gather(e50, fea50, idx2, xidx2)
    g4 = g.reshape(T * B, SLOTS_PAD, L11)
    xg3 = xg.reshape(T * B, D)

    in_specs, out_specs = _attn_specs()
    wzt = Wz.T.astype(f32)
    wr1t = Wr1.T.astype(f32)
    args = (
        g4, xg3,
        jnp.stack([w1_h1[:L11], w1_h1[L11:]], axis=1).astype(f32),
        w1_h0[:M].reshape(1, M), w1_h0[M:].reshape(1, M),
        weights_hops_1.T[:L11].astype(f32),
        weights_hops_1.T[L11:].astype(f32),
        V1_h0.T.astype(f32),
        wzt[:M], wzt[M:M + D], wzt[M + D:],
        wr1t[:M], wr1t[M:M + D], wr1t[M + D:],
        Wr2_1.T.astype(f32),
        Wr2_2.T[:D].astype(f32), Wr2_2.T[D:].astype(f32),
        Wh_1.T.astype(f32),
        Wh_2.T[:D].astype(f32), Wh_2.T[D:].astype(f32),
        bz.reshape(1, M), br1.reshape(1, M), br2.reshape(1, D),
        bh.reshape(1, M),
    )
    h = pl.pallas_call(
        _attn_gru_body,
        grid=(B // BB, T),
        in_specs=in_specs,
        out_specs=out_specs,
        out_shape=jax.ShapeDtypeStruct((B, M), f32),
        scratch_shapes=[pltpu.VMEM((BB, M), f32)],
    )(*args)
    return h


# speedup vs baseline: 1.0746x; 1.0746x over previous
import functools

import jax
import jax.numpy as jnp
from jax import lax
from jax.experimental import pallas as pl
from jax.experimental.pallas import tpu as pltpu
from jax.experimental.pallas import tpu_sc as plsc

N_NODES = 50000
T = 3
D = 128
M = 128
L11 = 64
B = 1024
S1 = 10
S2 = 25
SLOTS = 1 + S1 + S1 * S2
SLOTS_PAD = 264
EROW = T * L11
FROW = T * D

NW = 32
TGT_PER_W = B // NW
PER_W = TGT_PER_W * SLOTS_PAD
CHUNK = 128
NCHUNK = PER_W // CHUNK

PROJ_BN = 2000
BB = 16


def _proj_body(f_ref, v_ref, ep_ref, fl_ref):
    f3 = f_ref[...]
    fl_ref[...] = f3.reshape(PROJ_BN * T, D)
    es = [jnp.dot(f3[:, t, :], v_ref[...], preferred_element_type=jnp.float32)
          for t in range(T)]
    ep_ref[...] = jnp.concatenate(es, axis=-1)


def _softmax(x):
    m = jnp.max(x, axis=-1, keepdims=True)
    e = jnp.exp(x - m)
    return e / jnp.sum(e, axis=-1, keepdims=True)


def _leaky(x):
    return jnp.where(x >= 0, x, 0.01 * x)


def _attn_gru_body(g_ref, xg_ref, wab_ref, w0a_ref, w0b_ref,
                   wha_ref, whb_ref, v0t_ref, wz1_ref, wz2_ref, wz3_ref,
                   wr11_ref, wr12_ref, wr13_ref, wr21t_ref,
                   wr22at_ref, wr22bt_ref, wh1t_ref, wh2at_ref, wh2bt_ref,
                   bz_ref, br1_ref, br2_ref, bh_ref, o_ref, h_ref):
    t = pl.program_id(1)
    et = g_ref[...].reshape(BB, SLOTS_PAD, L11)
    xt = xg_ref[...]
    w0a = w0a_ref[...]
    w0b = w0b_ref[...]
    wha = wha_ref[...]
    whb = whb_ref[...]
    v0t = v0t_ref[...]

    @pl.when(t == 0)
    def _():
        h_ref[...] = jnp.zeros((BB, M), jnp.float32)

    h = h_ref[...]

    ab3 = jax.lax.dot_general(
        et, wab_ref[...], (((2,), (0,)), ((), ())),
        preferred_element_type=jnp.float32)
    a_s = ab3[:, :, 0]
    b_s = ab3[:, :, 1]

    e_tgt = et[:, 0, :]
    e_h1 = et[:, 1:1 + S1, :]
    e_h2r = et[:, 1 + S1:1 + S1 + S1 * S2, :].reshape(BB, S1, S2, L11)

    a0 = a_s[:, 0]
    a1 = a_s[:, 1:1 + S1]
    b1 = b_s[:, 1:1 + S1]
    s2d = b_s[:, 1 + S1:1 + S1 + S1 * S2].reshape(BB, S1, S2)

    beta1 = _softmax(_leaky(a1[:, :, None] + s2d))
    agg1 = jnp.sum(beta1[..., None] * e_h2r, axis=2)
    e_h1f = e_h1.reshape(BB * S1, L11)
    agg1f = agg1.reshape(BB * S1, L11)
    hop1f = jax.nn.sigmoid(
        jnp.dot(e_h1f, wha, preferred_element_type=jnp.float32)
        + jnp.dot(agg1f, whb, preferred_element_type=jnp.float32))

    beta0 = _softmax(_leaky(a0[:, None] + b1))
    agg0 = jnp.sum(beta0[..., None] * e_h1, axis=1)
    hop0 = jax.nn.sigmoid(
        jnp.dot(e_tgt, wha, preferred_element_type=jnp.float32)
        + jnp.dot(agg0, whb, preferred_element_type=jnp.float32))

    p0 = jnp.dot(hop0, v0t, preferred_element_type=jnp.float32)
    p1f = jnp.dot(hop1f, v0t, preferred_element_type=jnp.float32)
    p1 = p1f.reshape(BB, S1, M)
    c0 = jnp.sum(p0 * w0a, axis=-1)
    c1 = jnp.sum(p1 * w0b[:, None, :], axis=-1)
    betac = _softmax(_leaky(c0[:, None] + c1))
    xnt = jnp.sum(betac[..., None] * p1, axis=1)

    z = jax.nn.sigmoid(
        jnp.dot(h, wz1_ref[...], preferred_element_type=jnp.float32)
        + jnp.dot(xt, wz2_ref[...], preferred_element_type=jnp.float32)
        + jnp.dot(xnt, wz3_ref[...], preferred_element_type=jnp.float32)
        + bz_ref[...])
    r1 = jax.nn.sigmoid(
        jnp.dot(h, wr11_ref[...], preferred_element_type=jnp.float32)
        + jnp.dot(xt, wr12_ref[...], preferred_element_type=jnp.float32)
        + jnp.dot(xnt, wr13_ref[...], preferred_element_type=jnp.float32)
        + br1_ref[...])
    r2 = jax.nn.sigmoid(
        jnp.dot(h, wr21t_ref[...], preferred_element_type=jnp.float32)
        + jnp.dot(xt, wr22at_ref[...], preferred_element_type=jnp.float32)
        + jnp.dot(xnt, wr22bt_ref[...], preferred_element_type=jnp.float32)
        + br2_ref[...])
    h_til = jnp.tanh(
        jnp.dot(r1 * h, wh1t_ref[...], preferred_element_type=jnp.float32)
        + jnp.dot(xt, wh2at_ref[...], preferred_element_type=jnp.float32)
        + jnp.dot(r2 * xnt, wh2bt_ref[...],
                  preferred_element_type=jnp.float32)
        + bh_ref[...])
    h = (1.0 - z) * h + z * h_til
    h_ref[...] = h
    o_ref[...] = h


def _full(shape):
    return pl.BlockSpec(shape, lambda i, t: (0,) * len(shape))


def _attn_specs():
    in_specs = [
        pl.BlockSpec((BB * SLOTS_PAD, L11),
                     lambda i, t: (t * (B // BB) + i, 0)),
        pl.BlockSpec((BB, D), lambda i, t: (t * (B // BB) + i, 0)),
        _full((L11, 2)),
        _full((1, M)), _full((1, M)),
        _full((L11, L11)), _full((L11, L11)),
        _full((L11, M)),
        _full((M, M)), _full((D, M)), _full((D, M)),
        _full((M, M)), _full((D, M)), _full((D, M)),
        _full((M, M)), _full((D, M)), _full((D, M)),
        _full((M, M)), _full((D, M)), _full((D, M)),
        _full((1, M)), _full((1, M)), _full((1, D)), _full((1, M)),
    ]
    out_specs = pl.BlockSpec((BB, M), lambda i, t: (i, 0))
    return in_specs, out_specs


def _sc_gather(e50, fea50, idx2, xidx2):
    try:
        info = plsc.get_sparse_core_info()
        nc, ns = info.num_cores, info.num_subcores
    except Exception:
        nc, ns = 2, 16
    mesh = plsc.VectorSubcoreMesh(core_axis_name="c", subcore_axis_name="s")

    @functools.partial(
        pl.kernel,
        out_type=(jax.ShapeDtypeStruct((T, B * SLOTS_PAD, L11), jnp.float32),
                  jax.ShapeDtypeStruct((T, B, D), jnp.float32)),
        mesh=mesh,
        scratch_types=[
            pltpu.VMEM((PER_W,), jnp.int32),
            pltpu.VMEM((CHUNK, EROW), jnp.float32),
            pltpu.VMEM((CHUNK, EROW), jnp.float32),
            pltpu.VMEM((TGT_PER_W,), jnp.int32),
            pltpu.VMEM((TGT_PER_W, FROW), jnp.float32),
            pltpu.SemaphoreType.DMA,
        ],
        compiler_params=pltpu.CompilerParams(use_tc_tiling_on_sc=False),
    )
    def k(e_hbm, fea_hbm, idx_hbm, xidx_hbm, g_hbm, xg_hbm,
          idx_v, buf0, buf1, xidx_v, xrows_v, sem):
        wid = lax.axis_index("s") * nc + lax.axis_index("c")
        pltpu.sync_copy(idx_hbm.at[pl.ds(wid * PER_W, PER_W)], idx_v)
        pltpu.sync_copy(xidx_hbm.at[pl.ds(wid * TGT_PER_W, TGT_PER_W)],
                        xidx_v)
        pltpu.async_copy(fea_hbm.at[xidx_v], xrows_v, sem).wait()
        xoff = pl.multiple_of(wid * TGT_PER_W, 8)
        for tt in range(T):
            pltpu.sync_copy(xrows_v.at[:, pl.ds(tt * D, D)],
                            xg_hbm.at[tt, pl.ds(xoff, TGT_PER_W)])
        base = wid * PER_W
        bufs = (buf0, buf1)

        def fire(c, buf):
            off = pl.multiple_of(c * CHUNK, 8)
            pltpu.async_copy(e_hbm.at[idx_v.at[pl.ds(off, CHUNK)]], buf, sem)

        def drain(buf):
            pltpu.make_async_copy(e_hbm.at[pl.ds(0, CHUNK)], buf, sem).wait()

        fire(0, buf0)

        @pl.loop(0, NCHUNK, step=2)
        def _(j):
            for b in range(2):
                cur = j + b

                @pl.when(cur + 1 < NCHUNK)
                def _():
                    fire(cur + 1, bufs[1 - b])

                drain(bufs[b])
                off = pl.multiple_of(base + cur * CHUNK, 8)
                for tt in range(T):
                    pltpu.sync_copy(bufs[b].at[:, pl.ds(tt * L11, L11)],
                                    g_hbm.at[tt, pl.ds(off, CHUNK)])

    return k(e50, fea50, idx2, xidx2)


def kernel(x, saps_idx, Fea, V1_h1, w1_h1, V1_h0, w1_h0, weights_hops_1,
           Wz, Wr1, Wr2_1, Wr2_2, Wh_1, Wh_2, bz, br1, br2, bh):
    f32 = jnp.float32

    e_pack, fea_lin = pl.pallas_call(
        _proj_body,
        grid=(N_NODES // PROJ_BN,),
        in_specs=[pl.BlockSpec((PROJ_BN, T, D), lambda i: (i, 0, 0)),
                  pl.BlockSpec((D, L11), lambda i: (0, 0))],
        out_specs=[
            pl.BlockSpec((PROJ_BN, EROW), lambda i: (i, 0)),
            pl.BlockSpec((PROJ_BN * T, D), lambda i: (i, 0)),
        ],
        out_shape=[
            jax.ShapeDtypeStruct((N_NODES, EROW), f32),
            jax.ShapeDtypeStruct((N_NODES * T, D), f32),
        ],
    )(Fea, V1_h1.T.astype(f32))
    e50 = e_pack
    fea50 = fea_lin.reshape(N_NODES, FROW)

    idx_flat = jnp.concatenate(
        [saps_idx.astype(jnp.int32),
         jnp.zeros((B, SLOTS_PAD - SLOTS), jnp.int32)], axis=1).reshape(-1)
    xidx_flat = x.astype(jnp.int32)
    g, xg = _sc_gather(e50, fea50, idx_flat, xidx_flat)
    g4 = g.reshape(T * B * SLOTS_PAD, L11)
    xg3 = xg.reshape(T * B, D)

    in_specs, out_specs = _attn_specs()
    wzt = Wz.T.astype(f32)
    wr1t = Wr1.T.astype(f32)
    args = (
        g4, xg3,
        jnp.stack([w1_h1[:L11], w1_h1[L11:]], axis=1).astype(f32),
        w1_h0[:M].reshape(1, M), w1_h0[M:].reshape(1, M),
        weights_hops_1.T[:L11].astype(f32),
        weights_hops_1.T[L11:].astype(f32),
        V1_h0.T.astype(f32),
        wzt[:M], wzt[M:M + D], wzt[M + D:],
        wr1t[:M], wr1t[M:M + D], wr1t[M + D:],
        Wr2_1.T.astype(f32),
        Wr2_2.T[:D].astype(f32), Wr2_2.T[D:].astype(f32),
        Wh_1.T.astype(f32),
        Wh_2.T[:D].astype(f32), Wh_2.T[D:].astype(f32),
        bz.reshape(1, M), br1.reshape(1, M), br2.reshape(1, D),
        bh.reshape(1, M),
    )
    h = pl.pallas_call(
        _attn_gru_body,
        grid=(B // BB, T),
        in_specs=in_specs,
        out_specs=out_specs,
        out_shape=jax.ShapeDtypeStruct((B, M), f32),
        scratch_shapes=[pltpu.VMEM((BB, M), f32)],
    )(*args)
    return h


# speedup vs baseline: 1.1189x; 1.0412x over previous
import functools

import jax
import jax.numpy as jnp
from jax import lax
from jax.experimental import pallas as pl
from jax.experimental.pallas import tpu as pltpu
from jax.experimental.pallas import tpu_sc as plsc

N_NODES = 50000
T = 3
D = 128
M = 128
L11 = 64
B = 1024
S1 = 10
S2 = 25
SLOTS = 1 + S1 + S1 * S2
SLOTS_PAD = 264
EROW = T * L11
FROW = T * D

NW = 32
TGT_PER_W = B // NW
PER_W = TGT_PER_W * SLOTS_PAD
CHUNK = 128
NCHUNK = PER_W // CHUNK

PROJ_BN = 2000
BB = 32


def _proj_body(f_ref, v_ref, ep_ref):
    f3 = f_ref[...]
    es = [jnp.dot(f3[:, t, :], v_ref[...], preferred_element_type=jnp.float32)
          for t in range(T)]
    ep_ref[...] = jnp.concatenate(es, axis=-1)


def _softmax(x):
    m = jnp.max(x, axis=-1, keepdims=True)
    e = jnp.exp(x - m)
    return e / jnp.sum(e, axis=-1, keepdims=True)


def _leaky(x):
    return jnp.where(x >= 0, x, 0.01 * x)


def _attn_gru_body(g_ref, xg_ref, wab_ref, w0a_ref, w0b_ref,
                   wha_ref, whb_ref, v0t_ref, wz1_ref, wz2_ref, wz3_ref,
                   wr11_ref, wr12_ref, wr13_ref, wr21t_ref,
                   wr22at_ref, wr22bt_ref, wh1t_ref, wh2at_ref, wh2bt_ref,
                   bz_ref, br1_ref, br2_ref, bh_ref, o_ref, h_ref):
    t = pl.program_id(1)
    et = g_ref[...].reshape(BB, SLOTS_PAD, L11)
    xt = xg_ref[...]
    w0a = w0a_ref[...]
    w0b = w0b_ref[...]
    wha = wha_ref[...]
    whb = whb_ref[...]
    v0t = v0t_ref[...]

    @pl.when(t == 0)
    def _():
        h_ref[...] = jnp.zeros((BB, M), jnp.float32)

    h = h_ref[...]

    ab3 = jax.lax.dot_general(
        et, wab_ref[...], (((2,), (0,)), ((), ())),
        preferred_element_type=jnp.float32)
    a_s = ab3[:, :, 0]
    b_s = ab3[:, :, 1]

    e_tgt = et[:, 0, :]
    e_h1 = et[:, 1:1 + S1, :]
    e_h2r = et[:, 1 + S1:1 + S1 + S1 * S2, :].reshape(BB, S1, S2, L11)

    a0 = a_s[:, 0]
    a1 = a_s[:, 1:1 + S1]
    b1 = b_s[:, 1:1 + S1]
    s2d = b_s[:, 1 + S1:1 + S1 + S1 * S2].reshape(BB, S1, S2)

    beta1 = _softmax(_leaky(a1[:, :, None] + s2d))
    agg1 = jnp.sum(beta1[..., None] * e_h2r, axis=2)
    e_h1f = e_h1.reshape(BB * S1, L11)
    agg1f = agg1.reshape(BB * S1, L11)
    hop1f = jax.nn.sigmoid(
        jnp.dot(e_h1f, wha, preferred_element_type=jnp.float32)
        + jnp.dot(agg1f, whb, preferred_element_type=jnp.float32))

    beta0 = _softmax(_leaky(a0[:, None] + b1))
    agg0 = jnp.sum(beta0[..., None] * e_h1, axis=1)
    hop0 = jax.nn.sigmoid(
        jnp.dot(e_tgt, wha, preferred_element_type=jnp.float32)
        + jnp.dot(agg0, whb, preferred_element_type=jnp.float32))

    p0 = jnp.dot(hop0, v0t, preferred_element_type=jnp.float32)
    p1f = jnp.dot(hop1f, v0t, preferred_element_type=jnp.float32)
    p1 = p1f.reshape(BB, S1, M)
    c0 = jnp.sum(p0 * w0a, axis=-1)
    c1 = jnp.sum(p1 * w0b[:, None, :], axis=-1)
    betac = _softmax(_leaky(c0[:, None] + c1))
    xnt = jnp.sum(betac[..., None] * p1, axis=1)

    z = jax.nn.sigmoid(
        jnp.dot(h, wz1_ref[...], preferred_element_type=jnp.float32)
        + jnp.dot(xt, wz2_ref[...], preferred_element_type=jnp.float32)
        + jnp.dot(xnt, wz3_ref[...], preferred_element_type=jnp.float32)
        + bz_ref[...])
    r1 = jax.nn.sigmoid(
        jnp.dot(h, wr11_ref[...], preferred_element_type=jnp.float32)
        + jnp.dot(xt, wr12_ref[...], preferred_element_type=jnp.float32)
        + jnp.dot(xnt, wr13_ref[...], preferred_element_type=jnp.float32)
        + br1_ref[...])
    r2 = jax.nn.sigmoid(
        jnp.dot(h, wr21t_ref[...], preferred_element_type=jnp.float32)
        + jnp.dot(xt, wr22at_ref[...], preferred_element_type=jnp.float32)
        + jnp.dot(xnt, wr22bt_ref[...], preferred_element_type=jnp.float32)
        + br2_ref[...])
    h_til = jnp.tanh(
        jnp.dot(r1 * h, wh1t_ref[...], preferred_element_type=jnp.float32)
        + jnp.dot(xt, wh2at_ref[...], preferred_element_type=jnp.float32)
        + jnp.dot(r2 * xnt, wh2bt_ref[...],
                  preferred_element_type=jnp.float32)
        + bh_ref[...])
    h = (1.0 - z) * h + z * h_til
    h_ref[...] = h
    o_ref[...] = h


def _full(shape):
    return pl.BlockSpec(shape, lambda i, t: (0,) * len(shape))


def _attn_specs():
    in_specs = [
        pl.BlockSpec((BB * SLOTS_PAD, L11),
                     lambda i, t: (t * (B // BB) + i, 0)),
        pl.BlockSpec((BB, D), lambda i, t: (t * (B // BB) + i, 0)),
        _full((L11, 2)),
        _full((1, M)), _full((1, M)),
        _full((L11, L11)), _full((L11, L11)),
        _full((L11, M)),
        _full((M, M)), _full((D, M)), _full((D, M)),
        _full((M, M)), _full((D, M)), _full((D, M)),
        _full((M, M)), _full((D, M)), _full((D, M)),
        _full((M, M)), _full((D, M)), _full((D, M)),
        _full((1, M)), _full((1, M)), _full((1, D)), _full((1, M)),
    ]
    out_specs = pl.BlockSpec((BB, M), lambda i, t: (i, 0))
    return in_specs, out_specs


def _sc_gather(e50, fea50, idx2, xidx2):
    try:
        info = plsc.get_sparse_core_info()
        nc, ns = info.num_cores, info.num_subcores
    except Exception:
        nc, ns = 2, 16
    mesh = plsc.VectorSubcoreMesh(core_axis_name="c", subcore_axis_name="s")

    @functools.partial(
        pl.kernel,
        out_type=(jax.ShapeDtypeStruct((T, B * SLOTS_PAD, L11), jnp.float32),
                  jax.ShapeDtypeStruct((T, B, D), jnp.float32)),
        mesh=mesh,
        scratch_types=[
            pltpu.VMEM((PER_W,), jnp.int32),
            pltpu.VMEM((CHUNK, EROW), jnp.float32),
            pltpu.VMEM((CHUNK, EROW), jnp.float32),
            pltpu.VMEM((TGT_PER_W,), jnp.int32),
            pltpu.VMEM((TGT_PER_W, FROW), jnp.float32),
            pltpu.SemaphoreType.DMA,
        ],
        compiler_params=pltpu.CompilerParams(use_tc_tiling_on_sc=False),
    )
    def k(e_hbm, fea_hbm, idx_hbm, xidx_hbm, g_hbm, xg_hbm,
          idx_v, buf0, buf1, xidx_v, xrows_v, sem):
        wid = lax.axis_index("s") * nc + lax.axis_index("c")
        pltpu.sync_copy(idx_hbm.at[pl.ds(wid * PER_W, PER_W)], idx_v)
        pltpu.sync_copy(xidx_hbm.at[pl.ds(wid * TGT_PER_W, TGT_PER_W)],
                        xidx_v)
        pltpu.async_copy(fea_hbm.at[xidx_v], xrows_v, sem).wait()
        xoff = pl.multiple_of(wid * TGT_PER_W, 8)
        for tt in range(T):
            pltpu.sync_copy(xrows_v.at[:, pl.ds(tt * D, D)],
                            xg_hbm.at[tt, pl.ds(xoff, TGT_PER_W)])
        base = wid * PER_W
        bufs = (buf0, buf1)

        def fire(c, buf):
            off = pl.multiple_of(c * CHUNK, 8)
            pltpu.async_copy(e_hbm.at[idx_v.at[pl.ds(off, CHUNK)]], buf, sem)

        def drain(buf):
            pltpu.make_async_copy(e_hbm.at[pl.ds(0, CHUNK)], buf, sem).wait()

        fire(0, buf0)

        @pl.loop(0, NCHUNK, step=2)
        def _(j):
            for b in range(2):
                cur = j + b

                @pl.when(cur + 1 < NCHUNK)
                def _():
                    fire(cur + 1, bufs[1 - b])

                drain(bufs[b])
                off = pl.multiple_of(base + cur * CHUNK, 8)
                for tt in range(T):
                    pltpu.sync_copy(bufs[b].at[:, pl.ds(tt * L11, L11)],
                                    g_hbm.at[tt, pl.ds(off, CHUNK)])

    return k(e50, fea50, idx2, xidx2)


def kernel(x, saps_idx, Fea, V1_h1, w1_h1, V1_h0, w1_h0, weights_hops_1,
           Wz, Wr1, Wr2_1, Wr2_2, Wh_1, Wh_2, bz, br1, br2, bh):
    f32 = jnp.float32

    e_pack = pl.pallas_call(
        _proj_body,
        grid=(N_NODES // PROJ_BN,),
        in_specs=[pl.BlockSpec((PROJ_BN, T, D), lambda i: (i, 0, 0)),
                  pl.BlockSpec((D, L11), lambda i: (0, 0))],
        out_specs=pl.BlockSpec((PROJ_BN, EROW), lambda i: (i, 0)),
        out_shape=jax.ShapeDtypeStruct((N_NODES, EROW), f32),
    )(Fea, V1_h1.T.astype(f32))
    e50 = e_pack
    fea50 = Fea.astype(f32).reshape(N_NODES, FROW)

    idx_flat = jnp.concatenate(
        [saps_idx.astype(jnp.int32),
         jnp.zeros((B, SLOTS_PAD - SLOTS), jnp.int32)], axis=1).reshape(-1)
    xidx_flat = x.astype(jnp.int32)
    g, xg = _sc_gather(e50, fea50, idx_flat, xidx_flat)
    g4 = g.reshape(T * B * SLOTS_PAD, L11)
    xg3 = xg.reshape(T * B, D)

    in_specs, out_specs = _attn_specs()
    wzt = Wz.T.astype(f32)
    wr1t = Wr1.T.astype(f32)
    args = (
        g4, xg3,
        jnp.stack([w1_h1[:L11], w1_h1[L11:]], axis=1).astype(f32),
        w1_h0[:M].reshape(1, M), w1_h0[M:].reshape(1, M),
        weights_hops_1.T[:L11].astype(f32),
        weights_hops_1.T[L11:].astype(f32),
        V1_h0.T.astype(f32),
        wzt[:M], wzt[M:M + D], wzt[M + D:],
        wr1t[:M], wr1t[M:M + D], wr1t[M + D:],
        Wr2_1.T.astype(f32),
        Wr2_2.T[:D].astype(f32), Wr2_2.T[D:].astype(f32),
        Wh_1.T.astype(f32),
        Wh_2.T[:D].astype(f32), Wh_2.T[D:].astype(f32),
        bz.reshape(1, M), br1.reshape(1, M), br2.reshape(1, D),
        bh.reshape(1, M),
    )
    h = pl.pallas_call(
        _attn_gru_body,
        grid=(B // BB, T),
        in_specs=in_specs,
        out_specs=out_specs,
        out_shape=jax.ShapeDtypeStruct((B, M), f32),
        scratch_shapes=[pltpu.VMEM((BB, M), f32)],
    )(*args)
    return h


# speedup vs baseline: 1.1505x; 1.0282x over previous
import functools

import jax
import jax.numpy as jnp
from jax import lax
from jax.experimental import pallas as pl
from jax.experimental.pallas import tpu as pltpu
from jax.experimental.pallas import tpu_sc as plsc

N_NODES = 50000
T = 3
D = 128
M = 128
L11 = 64
B = 1024
S1 = 10
S2 = 25
SLOTS = 1 + S1 + S1 * S2
SLOTS_PAD = 264
EROW = T * L11
FROW = T * D

NW = 32
TGT_PER_W = B // NW
PER_W = TGT_PER_W * SLOTS_PAD
CHUNK = 128
NCHUNK = PER_W // CHUNK

PROJ_BN = 2000
BB = 64


def _proj_body(f_ref, v_ref, ep_ref):
    f3 = f_ref[...]
    es = [jnp.dot(f3[:, t, :], v_ref[...], preferred_element_type=jnp.float32)
          for t in range(T)]
    ep_ref[...] = jnp.concatenate(es, axis=-1)


def _softmax(x):
    m = jnp.max(x, axis=-1, keepdims=True)
    e = jnp.exp(x - m)
    return e / jnp.sum(e, axis=-1, keepdims=True)


def _leaky(x):
    return jnp.where(x >= 0, x, 0.01 * x)


def _attn_gru_body(g_ref, xg_ref, wab_ref, w0a_ref, w0b_ref,
                   wha_ref, whb_ref, v0t_ref, wz1_ref, wz2_ref, wz3_ref,
                   wr11_ref, wr12_ref, wr13_ref, wr21t_ref,
                   wr22at_ref, wr22bt_ref, wh1t_ref, wh2at_ref, wh2bt_ref,
                   bz_ref, br1_ref, br2_ref, bh_ref, o_ref, h_ref):
    t = pl.program_id(1)
    et = g_ref[...].reshape(BB, SLOTS_PAD, L11)
    xt = xg_ref[...]
    w0a = w0a_ref[...]
    w0b = w0b_ref[...]
    wha = wha_ref[...]
    whb = whb_ref[...]
    v0t = v0t_ref[...]

    @pl.when(t == 0)
    def _():
        h_ref[...] = jnp.zeros((BB, M), jnp.float32)

    h = h_ref[...]

    ab3 = jax.lax.dot_general(
        et, wab_ref[...], (((2,), (0,)), ((), ())),
        preferred_element_type=jnp.float32)
    a_s = ab3[:, :, 0]
    b_s = ab3[:, :, 1]

    e_tgt = et[:, 0, :]
    e_h1 = et[:, 1:1 + S1, :]
    e_h2r = et[:, 1 + S1:1 + S1 + S1 * S2, :].reshape(BB, S1, S2, L11)

    a0 = a_s[:, 0]
    a1 = a_s[:, 1:1 + S1]
    b1 = b_s[:, 1:1 + S1]
    s2d = b_s[:, 1 + S1:1 + S1 + S1 * S2].reshape(BB, S1, S2)

    beta1 = _softmax(_leaky(a1[:, :, None] + s2d))
    agg1 = jnp.sum(beta1[..., None] * e_h2r, axis=2)
    e_h1f = e_h1.reshape(BB * S1, L11)
    agg1f = agg1.reshape(BB * S1, L11)
    hop1f = jax.nn.sigmoid(
        jnp.dot(e_h1f, wha, preferred_element_type=jnp.float32)
        + jnp.dot(agg1f, whb, preferred_element_type=jnp.float32))

    beta0 = _softmax(_leaky(a0[:, None] + b1))
    agg0 = jnp.sum(beta0[..., None] * e_h1, axis=1)
    hop0 = jax.nn.sigmoid(
        jnp.dot(e_tgt, wha, preferred_element_type=jnp.float32)
        + jnp.dot(agg0, whb, preferred_element_type=jnp.float32))

    p0 = jnp.dot(hop0, v0t, preferred_element_type=jnp.float32)
    p1f = jnp.dot(hop1f, v0t, preferred_element_type=jnp.float32)
    p1 = p1f.reshape(BB, S1, M)
    c0 = jnp.sum(p0 * w0a, axis=-1)
    c1 = jnp.sum(p1 * w0b[:, None, :], axis=-1)
    betac = _softmax(_leaky(c0[:, None] + c1))
    xnt = jnp.sum(betac[..., None] * p1, axis=1)

    z = jax.nn.sigmoid(
        jnp.dot(h, wz1_ref[...], preferred_element_type=jnp.float32)
        + jnp.dot(xt, wz2_ref[...], preferred_element_type=jnp.float32)
        + jnp.dot(xnt, wz3_ref[...], preferred_element_type=jnp.float32)
        + bz_ref[...])
    r1 = jax.nn.sigmoid(
        jnp.dot(h, wr11_ref[...], preferred_element_type=jnp.float32)
        + jnp.dot(xt, wr12_ref[...], preferred_element_type=jnp.float32)
        + jnp.dot(xnt, wr13_ref[...], preferred_element_type=jnp.float32)
        + br1_ref[...])
    r2 = jax.nn.sigmoid(
        jnp.dot(h, wr21t_ref[...], preferred_element_type=jnp.float32)
        + jnp.dot(xt, wr22at_ref[...], preferred_element_type=jnp.float32)
        + jnp.dot(xnt, wr22bt_ref[...], preferred_element_type=jnp.float32)
        + br2_ref[...])
    h_til = jnp.tanh(
        jnp.dot(r1 * h, wh1t_ref[...], preferred_element_type=jnp.float32)
        + jnp.dot(xt, wh2at_ref[...], preferred_element_type=jnp.float32)
        + jnp.dot(r2 * xnt, wh2bt_ref[...],
                  preferred_element_type=jnp.float32)
        + bh_ref[...])
    h = (1.0 - z) * h + z * h_til
    h_ref[...] = h
    o_ref[...] = h


def _full(shape):
    return pl.BlockSpec(shape, lambda i, t: (0,) * len(shape))


def _attn_specs():
    in_specs = [
        pl.BlockSpec((BB * SLOTS_PAD, L11),
                     lambda i, t: (t * (B // BB) + i, 0)),
        pl.BlockSpec((BB, D), lambda i, t: (t * (B // BB) + i, 0)),
        _full((L11, 2)),
        _full((1, M)), _full((1, M)),
        _full((L11, L11)), _full((L11, L11)),
        _full((L11, M)),
        _full((M, M)), _full((D, M)), _full((D, M)),
        _full((M, M)), _full((D, M)), _full((D, M)),
        _full((M, M)), _full((D, M)), _full((D, M)),
        _full((M, M)), _full((D, M)), _full((D, M)),
        _full((1, M)), _full((1, M)), _full((1, D)), _full((1, M)),
    ]
    out_specs = pl.BlockSpec((BB, M), lambda i, t: (i, 0))
    return in_specs, out_specs


def _sc_gather(e50, fea50, idx2, xidx2):
    try:
        info = plsc.get_sparse_core_info()
        nc, ns = info.num_cores, info.num_subcores
    except Exception:
        nc, ns = 2, 16
    mesh = plsc.VectorSubcoreMesh(core_axis_name="c", subcore_axis_name="s")

    @functools.partial(
        pl.kernel,
        out_type=(jax.ShapeDtypeStruct((T, B * SLOTS_PAD, L11), jnp.float32),
                  jax.ShapeDtypeStruct((T, B, D), jnp.float32)),
        mesh=mesh,
        scratch_types=[
            pltpu.VMEM((PER_W,), jnp.int32),
            pltpu.VMEM((CHUNK, EROW), jnp.float32),
            pltpu.VMEM((CHUNK, EROW), jnp.float32),
            pltpu.VMEM((TGT_PER_W,), jnp.int32),
            pltpu.VMEM((TGT_PER_W, FROW), jnp.float32),
            pltpu.SemaphoreType.DMA,
        ],
        compiler_params=pltpu.CompilerParams(use_tc_tiling_on_sc=False),
    )
    def k(e_hbm, fea_hbm, idx_hbm, xidx_hbm, g_hbm, xg_hbm,
          idx_v, buf0, buf1, xidx_v, xrows_v, sem):
        wid = lax.axis_index("s") * nc + lax.axis_index("c")
        pltpu.sync_copy(idx_hbm.at[pl.ds(wid * PER_W, PER_W)], idx_v)
        pltpu.sync_copy(xidx_hbm.at[pl.ds(wid * TGT_PER_W, TGT_PER_W)],
                        xidx_v)
        pltpu.async_copy(fea_hbm.at[xidx_v], xrows_v, sem).wait()
        xoff = pl.multiple_of(wid * TGT_PER_W, 8)
        for tt in range(T):
            pltpu.sync_copy(xrows_v.at[:, pl.ds(tt * D, D)],
                            xg_hbm.at[tt, pl.ds(xoff, TGT_PER_W)])
        base = wid * PER_W
        bufs = (buf0, buf1)

        def fire(c, buf):
            off = pl.multiple_of(c * CHUNK, 8)
            pltpu.async_copy(e_hbm.at[idx_v.at[pl.ds(off, CHUNK)]], buf, sem)

        def drain(buf):
            pltpu.make_async_copy(e_hbm.at[pl.ds(0, CHUNK)], buf, sem).wait()

        fire(0, buf0)

        @pl.loop(0, NCHUNK, step=2)
        def _(j):
            for b in range(2):
                cur = j + b

                @pl.when(cur + 1 < NCHUNK)
                def _():
                    fire(cur + 1, bufs[1 - b])

                drain(bufs[b])
                off = pl.multiple_of(base + cur * CHUNK, 8)
                for tt in range(T):
                    pltpu.sync_copy(bufs[b].at[:, pl.ds(tt * L11, L11)],
                                    g_hbm.at[tt, pl.ds(off, CHUNK)])

    return k(e50, fea50, idx2, xidx2)


def kernel(x, saps_idx, Fea, V1_h1, w1_h1, V1_h0, w1_h0, weights_hops_1,
           Wz, Wr1, Wr2_1, Wr2_2, Wh_1, Wh_2, bz, br1, br2, bh):
    f32 = jnp.float32

    e_pack = pl.pallas_call(
        _proj_body,
        grid=(N_NODES // PROJ_BN,),
        in_specs=[pl.BlockSpec((PROJ_BN, T, D), lambda i: (i, 0, 0)),
                  pl.BlockSpec((D, L11), lambda i: (0, 0))],
        out_specs=pl.BlockSpec((PROJ_BN, EROW), lambda i: (i, 0)),
        out_shape=jax.ShapeDtypeStruct((N_NODES, EROW), f32),
    )(Fea, V1_h1.T.astype(f32))
    e50 = e_pack
    fea50 = Fea.astype(f32).reshape(N_NODES, FROW)

    idx_flat = jnp.concatenate(
        [saps_idx.astype(jnp.int32),
         jnp.zeros((B, SLOTS_PAD - SLOTS), jnp.int32)], axis=1).reshape(-1)
    xidx_flat = x.astype(jnp.int32)
    g, xg = _sc_gather(e50, fea50, idx_flat, xidx_flat)
    g4 = g.reshape(T * B * SLOTS_PAD, L11)
    xg3 = xg.reshape(T * B, D)

    in_specs, out_specs = _attn_specs()
    wzt = Wz.T.astype(f32)
    wr1t = Wr1.T.astype(f32)
    args = (
        g4, xg3,
        jnp.stack([w1_h1[:L11], w1_h1[L11:]], axis=1).astype(f32),
        w1_h0[:M].reshape(1, M), w1_h0[M:].reshape(1, M),
        weights_hops_1.T[:L11].astype(f32),
        weights_hops_1.T[L11:].astype(f32),
        V1_h0.T.astype(f32),
        wzt[:M], wzt[M:M + D], wzt[M + D:],
        wr1t[:M], wr1t[M:M + D], wr1t[M + D:],
        Wr2_1.T.astype(f32),
        Wr2_2.T[:D].astype(f32), Wr2_2.T[D:].astype(f32),
        Wh_1.T.astype(f32),
        Wh_2.T[:D].astype(f32), Wh_2.T[D:].astype(f32),
        bz.reshape(1, M), br1.reshape(1, M), br2.reshape(1, D),
        bh.reshape(1, M),
    )
    h = pl.pallas_call(
        _attn_gru_body,
        grid=(B // BB, T),
        in_specs=in_specs,
        out_specs=out_specs,
        out_shape=jax.ShapeDtypeStruct((B, M), f32),
        scratch_shapes=[pltpu.VMEM((BB, M), f32)],
    )(*args)
    return h
